# baseline (speedup 1.0000x reference)
import functools

import jax
import jax.numpy as jnp
from jax import lax
from jax.experimental import pallas as pl
from jax.experimental.pallas import tpu as pltpu
from jax.experimental.pallas import tpu_sc as plsc

D_MODEL = 64
GROUP = 128
NBUF = 5
SCALE = 8.0

_info = plsc.get_sparse_core_info()
_NC, _NS = _info.num_cores, _info.num_subcores
_NW = _NC * _NS


@functools.lru_cache(maxsize=None)
def _make_transpose(vocab: int):
    NWIN = vocab // 128
    TAIL = vocab - NWIN * 128
    NB2 = 6
    n_out2 = -(-(-(-NWIN // _NW)) // NB2)
    mesh = plsc.VectorSubcoreMesh(core_axis_name="c", subcore_axis_name="s")

    @functools.partial(
        pl.kernel,
        mesh=mesh,
        out_type=jax.ShapeDtypeStruct((vocab // 2, 128), jnp.float32),
        scratch_types=[
            pltpu.VMEM((NB2, D_MODEL, 128), jnp.float32),
            pltpu.VMEM((NB2, 64, 128), jnp.float32),
            pltpu.SemaphoreType.DMA((NB2,)),
            pltpu.SemaphoreType.DMA((NB2,)),
        ],
        compiler_params=pltpu.CompilerParams(use_tc_tiling_on_sc=True,
                                             needs_layout_passes=False),
    )
    def tr_kernel(tt_hbm, tail_hbm, y_hbm, in_v, out_v, gsem, ssem):
        wid = lax.axis_index("s") * _NC + lax.axis_index("c")
        lanes = lax.iota(jnp.int32, 16)
        perm = [lax.bitwise_and(lanes + k, 15) for k in range(16)]

        def in_descrs(b, w, nl=128):
            return [pltpu.make_async_copy(
                        tt_hbm.at[pl.ds(dt * 8, 8), pl.ds(w * 128, nl)],
                        in_v.at[b, pl.ds(dt * 8, 8), pl.ds(0, nl)],
                        gsem.at[b])
                    for dt in range(D_MODEL // 8)]

        def out_descr(b, w, nv=128):
            return pltpu.make_async_copy(out_v.at[b, pl.ds(0, nv // 2)],
                                         y_hbm.at[pl.ds(w * 64, nv // 2)],
                                         ssem.at[b])

        def transpose_blk(b, nvb):
            @plsc.parallel_loop(0, nvb * 4)
            def blk(i):
                v0 = lax.shift_right_logical(i, 2) * 16
                d0 = lax.bitwise_and(i, 3) * 16
                vvec = v0 + lanes
                pvec = lax.shift_right_logical(vvec, 1)
                lbase = lax.shift_left(lax.bitwise_and(vvec, 1), 6) + d0
                for k in range(16):
                    val = plsc.load_gather(in_v.at[b],
                                           [d0 + perm[k], vvec])
                    plsc.store_scatter(out_v.at[b],
                                       [pvec, lbase + perm[k]], val)

        for b in range(NB2):
            for d in in_descrs(b, wid + _NW * b):
                d.start()

        def outer(t, carry):
            for b in range(NB2):
                i_lin = t * NB2 + b
                w = wid + _NW * i_lin

                @pl.when(w < NWIN)
                def work():
                    for d in in_descrs(b, w):
                        d.wait()

                    @pl.when(i_lin >= NB2)
                    def drain_prev():
                        out_descr(b, w).wait()

                    transpose_blk(b, 8)
                    out_descr(b, w).start()

                    @pl.when(w + _NW * NB2 < NWIN)
                    def refill():
                        for d in in_descrs(b, w + _NW * NB2):
                            d.start()
            return carry

        lax.fori_loop(0, n_out2, outer, 0)
        for b in range(NB2):
            out_descr(b, 0).wait()

        if TAIL:
            @pl.when(wid == _NW - 1)
            def tail():
                stage = out_v.at[NB2 - 1, pl.ds(0, TAIL // 2)]
                pltpu.sync_copy(tail_hbm, stage)
                pltpu.sync_copy(
                    stage, y_hbm.at[pl.ds(vocab // 2 - TAIL // 2, TAIL // 2)])

    return tr_kernel


@functools.lru_cache(maxsize=None)
def _make_lookup(b_outer: int, seq: int):
    batch = b_outer * seq
    nbt = b_outer // GROUP
    n_groups_total = batch // GROUP
    n_groups = n_groups_total // _NW
    n_outer = n_groups // NBUF
    assert nbt == 32 and seq % 8 == 0 and n_groups % NBUF == 0
    mesh = plsc.VectorSubcoreMesh(core_axis_name="c", subcore_axis_name="s")

    @functools.partial(
        pl.kernel,
        mesh=mesh,
        out_type=jax.ShapeDtypeStruct((seq, 8, nbt, 8 * GROUP), jnp.float32),
        scratch_types=[
            pltpu.VMEM((n_groups, GROUP), jnp.int32),
            pltpu.VMEM((NBUF, GROUP, D_MODEL), jnp.float32),
            pltpu.VMEM((NBUF, D_MODEL * GROUP), jnp.float32),
            pltpu.SemaphoreType.DMA((NBUF,)),
            pltpu.SemaphoreType.DMA((NBUF,)),
        ],
        compiler_params=pltpu.CompilerParams(use_tc_tiling_on_sc=False,
                                             needs_layout_passes=False),
    )
    def emb_kernel(x_hbm, table_hbm, out_hbm, idx_v, rows_v, trans_v, gsem,
                   ssem):
        wid = lax.axis_index("s") * _NC + lax.axis_index("c")
        base = wid * n_groups
        pltpu.sync_copy(x_hbm.at[pl.ds(base, n_groups)], idx_v)

        lanes = lax.iota(jnp.int32, 16)
        perm = [lax.bitwise_and(lanes + k, 15) for k in range(16)]
        permg = [p * GROUP for p in perm]

        def store_descrs(b, g):
            gg = base + g
            s = lax.bitwise_or(
                lax.shift_left(lax.shift_right_logical(gg, 8), 3),
                lax.bitwise_and(gg, 7))
            bt = lax.bitwise_and(lax.shift_right_logical(gg, 3), nbt - 1)
            return [pltpu.make_async_copy(
                        trans_v.at[b, pl.ds(dt * 8 * GROUP, 8 * GROUP)],
                        out_hbm.at[s, dt, bt], ssem.at[b])
                    for dt in range(D_MODEL // 8)]

        for b in range(NBUF):
            pltpu.async_copy(table_hbm.at[idx_v.at[b]], rows_v.at[b],
                             gsem.at[b])

        def outer(t, carry):
            for b in range(NBUF):
                g = t * NBUF + b
                pb = (b - 1) % NBUF
                pg = g - 1 + NBUF

                @pl.when((g > 0) & (pg < n_groups))
                def refill():
                    for d in store_descrs(pb, pg - NBUF):
                        d.wait()
                    pltpu.async_copy(table_hbm.at[idx_v.at[pg]],
                                     rows_v.at[pb], gsem.at[pb])

                pltpu.make_async_copy(table_hbm.at[idx_v.at[g]],
                                      rows_v.at[b], gsem.at[b]).wait()

                nblk = (GROUP // 16) * (D_MODEL // 16)

                @plsc.parallel_loop(0, nblk)
                def scale_blk(i):
                    rvec = lax.shift_right_logical(i, 2) * 16 + lanes
                    d0 = lax.bitwise_and(i, 3) * 16
                    sbase = d0 * GROUP + rvec
                    for k in range(16):
                        v = plsc.load_gather(rows_v.at[b],
                                             [rvec, d0 + perm[k]])
                        plsc.store_scatter(trans_v.at[b],
                                           [sbase + permg[k]], v * SCALE)

                for d in store_descrs(b, g):
                    d.start()
            return carry

        lax.fori_loop(0, n_outer, outer, 0)
        for b in range(NBUF):
            g_last = n_groups - NBUF + b
            for d in store_descrs(b, g_last):
                d.wait()

    return emb_kernel


def kernel(x, table):
    b0, b1 = x.shape
    vocab = table.shape[0]
    nbt = b0 // GROUP
    x2d = (x.T.reshape(b1 // 8, 8, nbt, GROUP)
           .transpose(0, 2, 1, 3).reshape((b0 * b1) // GROUP, GROUP))
    tail_n = vocab - (vocab // 128) * 128
    tail32 = table[vocab - tail_n:, :].reshape(tail_n // 2, 128)
    y = _make_transpose(vocab)(table.T, tail32)
    out4 = _make_lookup(b0, b1)(x2d, y.reshape(vocab, D_MODEL))
    return (out4.reshape(b1, 8, nbt, 8, GROUP)
            .transpose(2, 4, 0, 1, 3).reshape(b0, b1, D_MODEL))


# speedup vs baseline: 1.1291x; 1.1291x over previous
import functools

import jax
import jax.numpy as jnp
from jax import lax
from jax.experimental import pallas as pl
from jax.experimental.pallas import tpu as pltpu
from jax.experimental.pallas import tpu_sc as plsc

D_MODEL = 64
GROUP = 128
NBUF = 8
NTR = 4
SCALE = 8.0

_info = plsc.get_sparse_core_info()
_NC, _NS = _info.num_cores, _info.num_subcores
_NW = _NC * _NS


@functools.lru_cache(maxsize=None)
def _make_transpose(vocab: int):
    NWIN = vocab // 128
    TAIL = vocab - NWIN * 128
    NB2 = 4
    n_out2 = -(-(-(-NWIN // _NW)) // NB2)
    mesh = plsc.VectorSubcoreMesh(core_axis_name="c", subcore_axis_name="s")

    @functools.partial(
        pl.kernel,
        mesh=mesh,
        out_type=jax.ShapeDtypeStruct((vocab // 2, 128), jnp.float32),
        scratch_types=[
            pltpu.VMEM((NB2, D_MODEL, 128), jnp.float32),
            pltpu.VMEM((NB2, 64, 128), jnp.float32),
            pltpu.SemaphoreType.DMA((NB2,)),
            pltpu.SemaphoreType.DMA((NB2,)),
        ],
        compiler_params=pltpu.CompilerParams(use_tc_tiling_on_sc=True,
                                             needs_layout_passes=False),
    )
    def tr_kernel(tt_hbm, tail_hbm, y_hbm, in_v, out_v, gsem, ssem):
        wid = lax.axis_index("s") * _NC + lax.axis_index("c")
        lanes = lax.iota(jnp.int32, 16)
        perm = [lax.bitwise_and(lanes + k, 15) for k in range(16)]

        def in_descrs(b, w, nl=128):
            return [pltpu.make_async_copy(
                        tt_hbm.at[pl.ds(dt * 8, 8), pl.ds(w * 128, nl)],
                        in_v.at[b, pl.ds(dt * 8, 8), pl.ds(0, nl)],
                        gsem.at[b])
                    for dt in range(D_MODEL // 8)]

        def out_descr(b, w, nv=128):
            return pltpu.make_async_copy(out_v.at[b, pl.ds(0, nv // 2)],
                                         y_hbm.at[pl.ds(w * 64, nv // 2)],
                                         ssem.at[b])

        def transpose_blk(b, nvb):
            @plsc.parallel_loop(0, nvb * 4)
            def blk(i):
                v0 = lax.shift_right_logical(i, 2) * 16
                d0 = lax.bitwise_and(i, 3) * 16
                vvec = v0 + lanes
                pvec = lax.shift_right_logical(vvec, 1)
                lbase = lax.shift_left(lax.bitwise_and(vvec, 1), 6) + d0
                for k in range(16):
                    val = plsc.load_gather(in_v.at[b],
                                           [d0 + perm[k], vvec])
                    plsc.store_scatter(out_v.at[b],
                                       [pvec, lbase + perm[k]], val)

        for b in range(NB2):
            for d in in_descrs(b, wid + _NW * b):
                d.start()

        def outer(t, carry):
            for b in range(NB2):
                i_lin = t * NB2 + b
                w = wid + _NW * i_lin

                @pl.when(w < NWIN)
                def work():
                    for d in in_descrs(b, w):
                        d.wait()

                    @pl.when(i_lin >= NB2)
                    def drain_prev():
                        out_descr(b, w).wait()

                    transpose_blk(b, 8)
                    out_descr(b, w).start()

                    @pl.when(w + _NW * NB2 < NWIN)
                    def refill():
                        for d in in_descrs(b, w + _NW * NB2):
                            d.start()
            return carry

        lax.fori_loop(0, n_out2, outer, 0)
        for b in range(NB2):
            out_descr(b, 0).wait()

        if TAIL:
            @pl.when(wid == _NW - 1)
            def tail():
                stage = out_v.at[NB2 - 1, pl.ds(0, TAIL // 2)]
                pltpu.sync_copy(tail_hbm, stage)
                pltpu.sync_copy(
                    stage, y_hbm.at[pl.ds(vocab // 2 - TAIL // 2, TAIL // 2)])

    return tr_kernel


@functools.lru_cache(maxsize=None)
def _make_lookup(b_outer: int, seq: int):
    batch = b_outer * seq
    nbt = b_outer // GROUP
    n_groups_total = batch // GROUP
    n_groups = n_groups_total // _NW
    n_outer = n_groups // NBUF
    assert nbt == 32 and seq % 8 == 0 and n_groups % NBUF == 0
    mesh = plsc.VectorSubcoreMesh(core_axis_name="c", subcore_axis_name="s")

    @functools.partial(
        pl.kernel,
        mesh=mesh,
        out_type=jax.ShapeDtypeStruct((seq, 8, nbt, 8 * GROUP), jnp.float32),
        scratch_types=[
            pltpu.VMEM((n_groups, GROUP), jnp.int32),
            pltpu.VMEM((NBUF, GROUP, D_MODEL), jnp.float32),
            pltpu.VMEM((NTR, D_MODEL * GROUP), jnp.float32),
            pltpu.SemaphoreType.DMA((NBUF,)),
            pltpu.SemaphoreType.DMA((NTR,)),
        ],
        compiler_params=pltpu.CompilerParams(use_tc_tiling_on_sc=False,
                                             needs_layout_passes=False),
    )
    def emb_kernel(x_hbm, table_hbm, out_hbm, idx_v, rows_v, trans_v, gsem,
                   ssem):
        wid = lax.axis_index("s") * _NC + lax.axis_index("c")
        base = wid * n_groups
        pltpu.sync_copy(x_hbm.at[pl.ds(base, n_groups)], idx_v)

        lanes = lax.iota(jnp.int32, 16)
        perm = [lax.bitwise_and(lanes + k, 15) for k in range(16)]
        permg = [p * GROUP for p in perm]

        def store_descrs(b, g):
            gg = base + g
            s = lax.bitwise_or(
                lax.shift_left(lax.shift_right_logical(gg, 8), 3),
                lax.bitwise_and(gg, 7))
            bt = lax.bitwise_and(lax.shift_right_logical(gg, 3), nbt - 1)
            return [pltpu.make_async_copy(
                        trans_v.at[b, pl.ds(dt * 8 * GROUP, 8 * GROUP)],
                        out_hbm.at[s, dt, bt], ssem.at[b])
                    for dt in range(D_MODEL // 8)]

        for b in range(NBUF):
            pltpu.async_copy(table_hbm.at[idx_v.at[b]], rows_v.at[b],
                             gsem.at[b])

        nblk = (GROUP // 16) * (D_MODEL // 16)

        def outer(t, carry):
            for b in range(NBUF):
                g = t * NBUF + b
                bt = b % NTR

                pltpu.make_async_copy(table_hbm.at[idx_v.at[g]],
                                      rows_v.at[b], gsem.at[b]).wait()

                @pl.when(g >= NTR)
                def drain_prev():
                    for d in store_descrs(bt, g):
                        d.wait()

                @plsc.parallel_loop(0, nblk)
                def scale_blk(i):
                    rvec = lax.shift_right_logical(i, 2) * 16 + lanes
                    d0 = lax.bitwise_and(i, 3) * 16
                    sbase = d0 * GROUP + rvec
                    for k in range(16):
                        v = plsc.load_gather(rows_v.at[b],
                                             [rvec, d0 + perm[k]])
                        plsc.store_scatter(trans_v.at[bt],
                                           [sbase + permg[k]], v * SCALE)

                for d in store_descrs(bt, g):
                    d.start()

                @pl.when(g < n_groups - NBUF)
                def refill():
                    pltpu.async_copy(table_hbm.at[idx_v.at[g + NBUF]],
                                     rows_v.at[b], gsem.at[b])
            return carry

        lax.fori_loop(0, n_outer, outer, 0)
        for bt in range(NTR):
            for d in store_descrs(bt, 0):
                d.wait()

    return emb_kernel


def kernel(x, table):
    b0, b1 = x.shape
    vocab = table.shape[0]
    nbt = b0 // GROUP
    x2d = (x.T.reshape(b1 // 8, 8, nbt, GROUP)
           .transpose(0, 2, 1, 3).reshape((b0 * b1) // GROUP, GROUP))
    tail_n = vocab - (vocab // 128) * 128
    tail32 = table[vocab - tail_n:, :].reshape(tail_n // 2, 128)
    y = _make_transpose(vocab)(table.T, tail32)
    out4 = _make_lookup(b0, b1)(x2d, y.reshape(vocab, D_MODEL))
    return (out4.reshape(b1, 8, nbt, 8, GROUP)
            .transpose(2, 4, 0, 1, 3).reshape(b0, b1, D_MODEL))
